# Initial kernel scaffold; baseline (speedup 1.0000x reference)
import functools

import jax
import jax.numpy as jnp
from jax import lax
from jax.experimental import pallas as pl
from jax.experimental.pallas import tpu as pltpu
from jax.experimental.pallas import tpu_sc as plsc

N_NODES = 10000
IN_DIM = 256
HID_DIM = 128
OUT_DIM = 64
N_EDGES = 160000

NC = 2
NS = 16
NW = NC * NS
EDGE_BLK = 128
EDGES_PER_TILE = -(-N_EDGES // NW)
BLOCKS = -(-EDGES_PER_TILE // EDGE_BLK)
EDGES_PAD = NW * BLOCKS * EDGE_BLK
ACC_ROWS = 10240
ROWS_PER_TILE = ACC_ROWS // NS
ZCHUNK = 128
CNT_W = 16

_mesh = plsc.VectorSubcoreMesh(core_axis_name="c", subcore_axis_name="s")


def _make_seg_sum(width: int, with_counts: bool):
  out_types = [jax.ShapeDtypeStruct((NC, ACC_ROWS, width), jnp.float32)]
  scratch = [
      pltpu.VMEM((BLOCKS, EDGE_BLK), jnp.int32),
      pltpu.VMEM((BLOCKS, EDGE_BLK), jnp.int32),
      pltpu.VMEM((EDGE_BLK, width), jnp.float32),
      pltpu.VMEM_SHARED((ACC_ROWS, width), jnp.float32),
  ]
  if with_counts:
    out_types.append(jax.ShapeDtypeStruct((NC, ACC_ROWS, CNT_W), jnp.float32))
    scratch += [
        pltpu.VMEM((EDGE_BLK, CNT_W), jnp.float32),
        pltpu.VMEM((ZCHUNK, CNT_W), jnp.float32),
        pltpu.VMEM_SHARED((ACC_ROWS, CNT_W), jnp.float32),
    ]

  def body(y_hbm, src_hbm, dst_hbm, *rest):
    if with_counts:
      (out_hbm, cnt_hbm, src_v, dst_v, gbuf, acc, ones_v, z16, cacc) = rest
    else:
      (out_hbm, src_v, dst_v, gbuf, acc) = rest
    c = lax.axis_index("c")
    s = lax.axis_index("s")
    wid = c * NS + s

    def fill_rows(i, _):
      for k in range(width // 16):
        gbuf[i, pl.ds(16 * k, 16)] = jnp.zeros((16,), jnp.float32)
      if with_counts:
        ones_v[i, :] = jnp.full((CNT_W,), 1.0, jnp.float32)
        z16[i, :] = jnp.zeros((CNT_W,), jnp.float32)
      return 0

    lax.fori_loop(0, EDGE_BLK, fill_rows, 0)

    base = s * ROWS_PER_TILE
    for k in range(ROWS_PER_TILE // ZCHUNK):
      pltpu.sync_copy(gbuf, acc.at[pl.ds(base + k * ZCHUNK, ZCHUNK)])
      if with_counts:
        pltpu.sync_copy(z16, cacc.at[pl.ds(base + k * ZCHUNK, ZCHUNK)])
    plsc.subcore_barrier()

    pltpu.sync_copy(src_hbm.at[wid], src_v)
    pltpu.sync_copy(dst_hbm.at[wid], dst_v)

    def step(j, _):
      pltpu.sync_copy(y_hbm.at[src_v.at[j]], gbuf)
      pltpu.sync_copy(gbuf, acc.at[dst_v.at[j]], add=True)
      if with_counts:
        pltpu.sync_copy(ones_v, cacc.at[dst_v.at[j]], add=True)
      return 0

    lax.fori_loop(0, BLOCKS, step, 0)
    plsc.subcore_barrier()

    pltpu.sync_copy(acc.at[pl.ds(base, ROWS_PER_TILE)],
                    out_hbm.at[c, pl.ds(base, ROWS_PER_TILE)])
    if with_counts:
      pltpu.sync_copy(cacc.at[pl.ds(base, ROWS_PER_TILE)],
                      cnt_hbm.at[c, pl.ds(base, ROWS_PER_TILE)])

  return pl.kernel(
      body,
      out_type=out_types,
      mesh=_mesh,
      scratch_types=scratch,
  )


_seg_sum_l1 = _make_seg_sum(HID_DIM, with_counts=True)
_seg_sum_l2 = _make_seg_sum(OUT_DIM, with_counts=False)

ROW_BLK = 1000
GRID = N_NODES // ROW_BLK


def _dot(a, b):
  return lax.dot_general(a, b, (((1,), (0,)), ((), ())),
                         precision=lax.Precision.HIGHEST,
                         preferred_element_type=jnp.float32)


def _stage_a_body(x_ref, wl_ref, wr_ref, y_ref, xr_ref):
  xb = x_ref[...]
  y_ref[...] = _dot(xb, wl_ref[...])
  xr_ref[...] = _dot(xb, wr_ref[...])


def _stage_a(x, w1l, w1r):
  return pl.pallas_call(
      _stage_a_body,
      grid=(GRID,),
      in_specs=[
          pl.BlockSpec((ROW_BLK, IN_DIM), lambda i: (i, 0)),
          pl.BlockSpec((IN_DIM, HID_DIM), lambda i: (0, 0)),
          pl.BlockSpec((IN_DIM, HID_DIM), lambda i: (0, 0)),
      ],
      out_specs=[
          pl.BlockSpec((ROW_BLK, HID_DIM), lambda i: (i, 0)),
          pl.BlockSpec((ROW_BLK, HID_DIM), lambda i: (i, 0)),
      ],
      out_shape=[
          jax.ShapeDtypeStruct((N_NODES, HID_DIM), jnp.float32),
          jax.ShapeDtypeStruct((N_NODES, HID_DIM), jnp.float32),
      ],
  )(x, w1l, w1r)


def _stage_c_body(p_ref, c_ref, xr_ref, b1_ref, w2l_ref, w2r_ref,
                  y2_ref, hr_ref):
  cnt = c_ref[0, :, 0:1] + c_ref[1, :, 0:1]
  rcp = 1.0 / jnp.maximum(cnt, 1.0)
  h = (p_ref[0] + p_ref[1]) * rcp + xr_ref[...] + b1_ref[...]
  h = jnp.maximum(h, 0.0)
  y2_ref[...] = _dot(h, w2l_ref[...])
  hr_ref[...] = _dot(h, w2r_ref[...])


def _stage_c(p, cnt, xr, b1, w2l, w2r):
  return pl.pallas_call(
      _stage_c_body,
      grid=(GRID,),
      in_specs=[
          pl.BlockSpec((NC, ROW_BLK, HID_DIM), lambda i: (0, i, 0)),
          pl.BlockSpec((NC, ROW_BLK, CNT_W), lambda i: (0, i, 0)),
          pl.BlockSpec((ROW_BLK, HID_DIM), lambda i: (i, 0)),
          pl.BlockSpec((1, HID_DIM), lambda i: (0, 0)),
          pl.BlockSpec((HID_DIM, OUT_DIM), lambda i: (0, 0)),
          pl.BlockSpec((HID_DIM, OUT_DIM), lambda i: (0, 0)),
      ],
      out_specs=[
          pl.BlockSpec((ROW_BLK, OUT_DIM), lambda i: (i, 0)),
          pl.BlockSpec((ROW_BLK, OUT_DIM), lambda i: (i, 0)),
      ],
      out_shape=[
          jax.ShapeDtypeStruct((N_NODES, OUT_DIM), jnp.float32),
          jax.ShapeDtypeStruct((N_NODES, OUT_DIM), jnp.float32),
      ],
  )(p, cnt, xr, b1, w2l, w2r)


def _stage_e_body(q_ref, c_ref, hr_ref, b2_ref, out_ref):
  cnt = c_ref[0, :, 0:1] + c_ref[1, :, 0:1]
  rcp = 1.0 / jnp.maximum(cnt, 1.0)
  out_ref[...] = (q_ref[0] + q_ref[1]) * rcp + hr_ref[...] + b2_ref[...]


def _stage_e(q, cnt, hr, b2):
  return pl.pallas_call(
      _stage_e_body,
      grid=(GRID,),
      in_specs=[
          pl.BlockSpec((NC, ROW_BLK, OUT_DIM), lambda i: (0, i, 0)),
          pl.BlockSpec((NC, ROW_BLK, CNT_W), lambda i: (0, i, 0)),
          pl.BlockSpec((ROW_BLK, OUT_DIM), lambda i: (i, 0)),
          pl.BlockSpec((1, OUT_DIM), lambda i: (0, 0)),
      ],
      out_specs=pl.BlockSpec((ROW_BLK, OUT_DIM), lambda i: (i, 0)),
      out_shape=jax.ShapeDtypeStruct((N_NODES, OUT_DIM), jnp.float32),
  )(q, cnt, hr, b2)


@jax.jit
def kernel(x, edge_index, W1_l, W1_r, b1, W2_l, W2_r, b2):
  src = edge_index[0].astype(jnp.int32)
  dst = edge_index[1].astype(jnp.int32)
  pad = EDGES_PAD - N_EDGES
  src3 = jnp.concatenate([src, jnp.zeros((pad,), jnp.int32)])
  src3 = src3.reshape(NW, BLOCKS, EDGE_BLK)
  dst3 = jnp.concatenate([dst, jnp.full((pad,), N_NODES, jnp.int32)])
  dst3 = dst3.reshape(NW, BLOCKS, EDGE_BLK)

  y1, xr = _stage_a(x, W1_l, W1_r)
  p1, c1 = _seg_sum_l1(y1, src3, dst3)
  y2, hr = _stage_c(p1, c1, xr, b1.reshape(1, HID_DIM), W2_l, W2_r)
  (q2,) = _seg_sum_l2(y2, src3, dst3)
  out = _stage_e(q2, c1, hr, b2.reshape(1, OUT_DIM))
  return out


# speedup vs baseline: 4.9596x; 4.9596x over previous
import jax
import jax.numpy as jnp
from jax import lax
from jax.experimental import pallas as pl
from jax.experimental.pallas import tpu as pltpu
from jax.experimental.pallas import tpu_sc as plsc

N_NODES = 10000
IN_DIM = 256
HID_DIM = 128
OUT_DIM = 64
N_EDGES = 160000

NC = 2
NS = 16
EDGE_BLK = 128
EDGES_PER_TILE = -(-N_EDGES // NS)
IBLOCKS = -(-EDGES_PER_TILE // EDGE_BLK)
IBLOCKS = -(-IBLOCKS // 8) * 8
EDGES_PAD = NS * IBLOCKS * EDGE_BLK
ICHUNK = 8
N_ICHUNKS = IBLOCKS // ICHUNK
ACC_ROWS = 10240
ROWS_PER_TILE = ACC_ROWS // NS
ZCHUNK = 128
CNT_W = 16

_mesh = plsc.VectorSubcoreMesh(core_axis_name="c", subcore_axis_name="s",
                               num_cores=NC, num_subcores=NS)


def _make_seg_sum(width: int, with_counts: bool):
  half = width // 2
  out_types = [jax.ShapeDtypeStruct((NC, ACC_ROWS, half), jnp.float32)]
  scratch = [
      pltpu.VMEM((ICHUNK, EDGE_BLK), jnp.int32),
      pltpu.VMEM((ICHUNK, EDGE_BLK), jnp.int32),
      pltpu.VMEM((EDGE_BLK, half), jnp.float32),
      pltpu.VMEM_SHARED((ACC_ROWS, half), jnp.float32),
  ]
  if with_counts:
    out_types.append(jax.ShapeDtypeStruct((ACC_ROWS, CNT_W), jnp.float32))
    scratch += [
        pltpu.VMEM((EDGE_BLK, CNT_W), jnp.float32),
        pltpu.VMEM((ZCHUNK, CNT_W), jnp.float32),
        pltpu.VMEM_SHARED((ACC_ROWS, CNT_W), jnp.float32),
    ]

  def body(y_hbm, src_hbm, dst_hbm, *rest):
    if with_counts:
      (out_hbm, cnt_hbm, src_v, dst_v, gbuf, acc, ones_v, z16, cacc) = rest
    else:
      (out_hbm, src_v, dst_v, gbuf, acc) = rest
    c = lax.axis_index("c")
    s = lax.axis_index("s")

    def fill_rows(i, _):
      for k in range(half // 16):
        gbuf[i, pl.ds(16 * k, 16)] = jnp.zeros((16,), jnp.float32)
      if with_counts:
        ones_v[i, :] = jnp.full((CNT_W,), 1.0, jnp.float32)
        z16[i, :] = jnp.zeros((CNT_W,), jnp.float32)
      return 0

    lax.fori_loop(0, EDGE_BLK, fill_rows, 0)

    base = s * ROWS_PER_TILE
    for k in range(ROWS_PER_TILE // ZCHUNK):
      pltpu.sync_copy(gbuf, acc.at[pl.ds(base + k * ZCHUNK, ZCHUNK)])
      if with_counts:
        @pl.when(c == 0)
        def _():
          pltpu.sync_copy(z16, cacc.at[pl.ds(base + k * ZCHUNK, ZCHUNK)])
    plsc.subcore_barrier()

    def chunk_body(ci, _):
      pltpu.sync_copy(src_hbm.at[c, s, pl.ds(ci * ICHUNK, ICHUNK)], src_v)
      pltpu.sync_copy(dst_hbm.at[s, pl.ds(ci * ICHUNK, ICHUNK)], dst_v)

      def step(j, _):
        pltpu.sync_copy(y_hbm.at[src_v.at[j]], gbuf)
        pltpu.sync_copy(gbuf, acc.at[dst_v.at[j]], add=True)
        if with_counts:
          @pl.when(c == 0)
          def _():
            pltpu.sync_copy(ones_v, cacc.at[dst_v.at[j]], add=True)
        return 0

      lax.fori_loop(0, ICHUNK, step, 0)
      return 0

    lax.fori_loop(0, N_ICHUNKS, chunk_body, 0)
    plsc.subcore_barrier()

    pltpu.sync_copy(acc.at[pl.ds(base, ROWS_PER_TILE)],
                    out_hbm.at[c, pl.ds(base, ROWS_PER_TILE)])
    if with_counts:
      @pl.when(c == 0)
      def _():
        pltpu.sync_copy(cacc.at[pl.ds(base, ROWS_PER_TILE)],
                        cnt_hbm.at[pl.ds(base, ROWS_PER_TILE)])

  return pl.kernel(
      body,
      out_type=out_types,
      mesh=_mesh,
      scratch_types=scratch,
      compiler_params=pltpu.CompilerParams(use_tc_tiling_on_sc=False),
  )


_seg_sum_l1 = _make_seg_sum(HID_DIM, with_counts=True)
_seg_sum_l2 = _make_seg_sum(OUT_DIM, with_counts=False)

ROW_BLK = 1000
GRID = N_NODES // ROW_BLK
H1 = HID_DIM // 2
H2 = OUT_DIM // 2


def _dot(a, b):
  return lax.dot_general(a, b, (((1,), (0,)), ((), ())),
                         precision=lax.Precision.HIGHEST,
                         preferred_element_type=jnp.float32)


def _stage_a_body(x_ref, wl_ref, wr_ref, y_ref, xr_ref):
  xb = x_ref[...]
  y = _dot(xb, wl_ref[...])
  y_ref[0] = y[:, :H1]
  y_ref[1] = y[:, H1:]
  xr_ref[...] = _dot(xb, wr_ref[...])


def _stage_a(x, w1l, w1r):
  return pl.pallas_call(
      _stage_a_body,
      grid=(GRID,),
      in_specs=[
          pl.BlockSpec((ROW_BLK, IN_DIM), lambda i: (i, 0)),
          pl.BlockSpec((IN_DIM, HID_DIM), lambda i: (0, 0)),
          pl.BlockSpec((IN_DIM, HID_DIM), lambda i: (0, 0)),
      ],
      out_specs=[
          pl.BlockSpec((NC, ROW_BLK, H1), lambda i: (0, i, 0)),
          pl.BlockSpec((ROW_BLK, HID_DIM), lambda i: (i, 0)),
      ],
      out_shape=[
          jax.ShapeDtypeStruct((NC, N_NODES, H1), jnp.float32),
          jax.ShapeDtypeStruct((N_NODES, HID_DIM), jnp.float32),
      ],
  )(x, w1l, w1r)


def _stage_c_body(p_ref, c_ref, xr_ref, b1_ref, w2l_ref, w2r_ref,
                  y2_ref, hr_ref):
  cnt = c_ref[:, 0:1]
  rcp = 1.0 / jnp.maximum(cnt, 1.0)
  mean = jnp.concatenate([p_ref[0], p_ref[1]], axis=1) * rcp
  h = mean + xr_ref[...] + b1_ref[...]
  h = jnp.maximum(h, 0.0)
  y2 = _dot(h, w2l_ref[...])
  y2_ref[0] = y2[:, :H2]
  y2_ref[1] = y2[:, H2:]
  hr_ref[...] = _dot(h, w2r_ref[...])


def _stage_c(p, cnt, xr, b1, w2l, w2r):
  return pl.pallas_call(
      _stage_c_body,
      grid=(GRID,),
      in_specs=[
          pl.BlockSpec((NC, ROW_BLK, H1), lambda i: (0, i, 0)),
          pl.BlockSpec((ROW_BLK, CNT_W), lambda i: (i, 0)),
          pl.BlockSpec((ROW_BLK, HID_DIM), lambda i: (i, 0)),
          pl.BlockSpec((1, HID_DIM), lambda i: (0, 0)),
          pl.BlockSpec((HID_DIM, OUT_DIM), lambda i: (0, 0)),
          pl.BlockSpec((HID_DIM, OUT_DIM), lambda i: (0, 0)),
      ],
      out_specs=[
          pl.BlockSpec((NC, ROW_BLK, H2), lambda i: (0, i, 0)),
          pl.BlockSpec((ROW_BLK, OUT_DIM), lambda i: (i, 0)),
      ],
      out_shape=[
          jax.ShapeDtypeStruct((NC, N_NODES, H2), jnp.float32),
          jax.ShapeDtypeStruct((N_NODES, OUT_DIM), jnp.float32),
      ],
  )(p, cnt, xr, b1, w2l, w2r)


def _stage_e_body(q_ref, c_ref, hr_ref, b2_ref, out_ref):
  cnt = c_ref[:, 0:1]
  rcp = 1.0 / jnp.maximum(cnt, 1.0)
  mean = jnp.concatenate([q_ref[0], q_ref[1]], axis=1) * rcp
  out_ref[...] = mean + hr_ref[...] + b2_ref[...]


def _stage_e(q, cnt, hr, b2):
  return pl.pallas_call(
      _stage_e_body,
      grid=(GRID,),
      in_specs=[
          pl.BlockSpec((NC, ROW_BLK, H2), lambda i: (0, i, 0)),
          pl.BlockSpec((ROW_BLK, CNT_W), lambda i: (i, 0)),
          pl.BlockSpec((ROW_BLK, OUT_DIM), lambda i: (i, 0)),
          pl.BlockSpec((1, OUT_DIM), lambda i: (0, 0)),
      ],
      out_specs=pl.BlockSpec((ROW_BLK, OUT_DIM), lambda i: (i, 0)),
      out_shape=jax.ShapeDtypeStruct((N_NODES, OUT_DIM), jnp.float32),
  )(q, cnt, hr, b2)


@jax.jit
def kernel(x, edge_index, W1_l, W1_r, b1, W2_l, W2_r, b2):
  src = edge_index[0].astype(jnp.int32)
  dst = edge_index[1].astype(jnp.int32)
  pad = EDGES_PAD - N_EDGES
  src_p = jnp.concatenate([src, jnp.zeros((pad,), jnp.int32)])
  src3 = jnp.stack([src_p, src_p + N_NODES]).reshape(NC, NS, IBLOCKS, EDGE_BLK)
  dst3 = jnp.concatenate([dst, jnp.full((pad,), N_NODES, jnp.int32)])
  dst3 = dst3.reshape(NS, IBLOCKS, EDGE_BLK)

  y1, xr = _stage_a(x, W1_l, W1_r)
  p1, c1 = _seg_sum_l1(y1.reshape(NC * N_NODES, H1), src3, dst3)
  y2, hr = _stage_c(p1, c1[:N_NODES], xr, b1.reshape(1, HID_DIM), W2_l, W2_r)
  (q2,) = _seg_sum_l2(y2.reshape(NC * N_NODES, H2), src3, dst3)
  out = _stage_e(q2, c1[:N_NODES], hr, b2.reshape(1, OUT_DIM))
  return out


# speedup vs baseline: 5.7677x; 1.1629x over previous
import jax
import jax.numpy as jnp
from jax import lax
from jax.experimental import pallas as pl
from jax.experimental.pallas import tpu as pltpu
from jax.experimental.pallas import tpu_sc as plsc

N_NODES = 10000
IN_DIM = 256
HID_DIM = 128
OUT_DIM = 64
N_EDGES = 160000

NC = 2
NS = 16
EDGE_BLK = 128
EDGES_PER_TILE = -(-N_EDGES // NS)
IBLOCKS = -(-EDGES_PER_TILE // EDGE_BLK)
IBLOCKS = -(-IBLOCKS // 8) * 8
EDGES_PAD = NS * IBLOCKS * EDGE_BLK
NBUF = 2
NGROUPS = IBLOCKS // NBUF
ACC_ROWS = 10240
ROWS_PER_TILE = ACC_ROWS // NS
ZCHUNK = 128
CNT_W = 16

_mesh = plsc.VectorSubcoreMesh(core_axis_name="c", subcore_axis_name="s",
                               num_cores=NC, num_subcores=NS)


def _make_seg_sum(width: int, with_counts: bool):
  half = width // 2
  out_types = [jax.ShapeDtypeStruct((NC, ACC_ROWS, half), jnp.float32)]
  scratch = [
      pltpu.VMEM((IBLOCKS, EDGE_BLK), jnp.int32),
      pltpu.VMEM((IBLOCKS, EDGE_BLK), jnp.int32),
      pltpu.VMEM((EDGE_BLK, half), jnp.float32),
      pltpu.VMEM((EDGE_BLK, half), jnp.float32),
      pltpu.VMEM_SHARED((ACC_ROWS, half), jnp.float32),
      pltpu.SemaphoreType.DMA,
      pltpu.SemaphoreType.DMA,
      pltpu.SemaphoreType.DMA,
      pltpu.SemaphoreType.DMA,
  ]
  if with_counts:
    out_types.append(jax.ShapeDtypeStruct((ACC_ROWS, CNT_W), jnp.float32))
    scratch += [
        pltpu.VMEM((EDGE_BLK, CNT_W), jnp.float32),
        pltpu.VMEM_SHARED((ACC_ROWS, CNT_W), jnp.float32),
        pltpu.SemaphoreType.DMA,
    ]

  def body(y_hbm, src_hbm, dst_hbm, zacc_hbm, *rest):
    if with_counts:
      (zcnt_hbm, ones_hbm, out_hbm, cnt_hbm, src_v, dst_v, gb0, gb1, acc,
       gsem0, gsem1, ssem0, ssem1, ones_v, cacc, csem) = rest
    else:
      (out_hbm, src_v, dst_v, gb0, gb1, acc,
       gsem0, gsem1, ssem0, ssem1) = rest
    gbuf = (gb0, gb1)
    gsem = (gsem0, gsem1)
    ssem = (ssem0, ssem1)
    c = lax.axis_index("c")
    s = lax.axis_index("s")
    base = s * ROWS_PER_TILE

    pltpu.sync_copy(src_hbm.at[c, s], src_v)
    pltpu.sync_copy(dst_hbm.at[s], dst_v)
    for k in range(ROWS_PER_TILE // ZCHUNK):
      pltpu.sync_copy(zacc_hbm, acc.at[pl.ds(base + k * ZCHUNK, ZCHUNK)])
    if with_counts:
      pltpu.sync_copy(ones_hbm, ones_v)

      @pl.when(c == 0)
      def _():
        for k in range(ROWS_PER_TILE // ZCHUNK):
          pltpu.sync_copy(zcnt_hbm, cacc.at[pl.ds(base + k * ZCHUNK, ZCHUNK)])
    plsc.subcore_barrier()

    for b in range(NBUF):
      pltpu.async_copy(y_hbm.at[src_v.at[b]], gbuf[b], gsem[b])

    def group(g, _):
      for b in range(NBUF):
        j = g * NBUF + b
        pltpu.make_async_copy(y_hbm.at[src_v.at[j]], gbuf[b], gsem[b]).wait()
        pltpu.async_copy(gbuf[b], acc.at[dst_v.at[j]], ssem[b], add=True)
        if with_counts:
          @pl.when(c == 0)
          def _():
            pltpu.async_copy(ones_v, cacc.at[dst_v.at[j]], csem, add=True)
      for b in range(NBUF):
        jn = (g + 1) * NBUF + b

        @pl.when(jn < IBLOCKS)
        def _():
          pltpu.make_async_copy(gbuf[b], acc.at[dst_v.at[jn]], ssem[b]).wait()
          pltpu.async_copy(y_hbm.at[src_v.at[jn]], gbuf[b], gsem[b])
      return 0

    lax.fori_loop(0, NGROUPS, group, 0)

    for b in range(NBUF):
      pltpu.make_async_copy(gbuf[b], acc.at[dst_v.at[0]], ssem[b]).wait()
    if with_counts:
      @pl.when(c == 0)
      def _():
        def drain(j, _):
          pltpu.make_async_copy(ones_v, cacc.at[dst_v.at[0]], csem).wait()
          return 0
        lax.fori_loop(0, IBLOCKS, drain, 0)
    plsc.subcore_barrier()

    pltpu.sync_copy(acc.at[pl.ds(base, ROWS_PER_TILE)],
                    out_hbm.at[c, pl.ds(base, ROWS_PER_TILE)])
    if with_counts:
      @pl.when(c == 0)
      def _():
        pltpu.sync_copy(cacc.at[pl.ds(base, ROWS_PER_TILE)],
                        cnt_hbm.at[pl.ds(base, ROWS_PER_TILE)])

  return pl.kernel(
      body,
      out_type=out_types,
      mesh=_mesh,
      scratch_types=scratch,
      compiler_params=pltpu.CompilerParams(use_tc_tiling_on_sc=False),
  )


_seg_sum_l1 = _make_seg_sum(HID_DIM, with_counts=True)
_seg_sum_l2 = _make_seg_sum(OUT_DIM, with_counts=False)

ROW_BLK = 1000
GRID = N_NODES // ROW_BLK
H1 = HID_DIM // 2
H2 = OUT_DIM // 2


def _dot(a, b):
  return lax.dot_general(a, b, (((1,), (0,)), ((), ())),
                         precision=lax.Precision.HIGHEST,
                         preferred_element_type=jnp.float32)


def _stage_a_body(x_ref, wl_ref, wr_ref, y_ref, xr_ref):
  xb = x_ref[...]
  y = _dot(xb, wl_ref[...])
  y_ref[0] = y[:, :H1]
  y_ref[1] = y[:, H1:]
  xr_ref[...] = _dot(xb, wr_ref[...])


def _stage_a(x, w1l, w1r):
  return pl.pallas_call(
      _stage_a_body,
      grid=(GRID,),
      in_specs=[
          pl.BlockSpec((ROW_BLK, IN_DIM), lambda i: (i, 0)),
          pl.BlockSpec((IN_DIM, HID_DIM), lambda i: (0, 0)),
          pl.BlockSpec((IN_DIM, HID_DIM), lambda i: (0, 0)),
      ],
      out_specs=[
          pl.BlockSpec((NC, ROW_BLK, H1), lambda i: (0, i, 0)),
          pl.BlockSpec((ROW_BLK, HID_DIM), lambda i: (i, 0)),
      ],
      out_shape=[
          jax.ShapeDtypeStruct((NC, N_NODES, H1), jnp.float32),
          jax.ShapeDtypeStruct((N_NODES, HID_DIM), jnp.float32),
      ],
  )(x, w1l, w1r)


def _stage_c_body(p_ref, c_ref, xr_ref, b1_ref, w2l_ref, w2r_ref,
                  y2_ref, hr_ref):
  cnt = c_ref[:, 0:1]
  rcp = 1.0 / jnp.maximum(cnt, 1.0)
  mean = jnp.concatenate([p_ref[0], p_ref[1]], axis=1) * rcp
  h = mean + xr_ref[...] + b1_ref[...]
  h = jnp.maximum(h, 0.0)
  y2 = _dot(h, w2l_ref[...])
  y2_ref[0] = y2[:, :H2]
  y2_ref[1] = y2[:, H2:]
  hr_ref[...] = _dot(h, w2r_ref[...])


def _stage_c(p, cnt, xr, b1, w2l, w2r):
  return pl.pallas_call(
      _stage_c_body,
      grid=(GRID,),
      in_specs=[
          pl.BlockSpec((NC, ROW_BLK, H1), lambda i: (0, i, 0)),
          pl.BlockSpec((ROW_BLK, CNT_W), lambda i: (i, 0)),
          pl.BlockSpec((ROW_BLK, HID_DIM), lambda i: (i, 0)),
          pl.BlockSpec((1, HID_DIM), lambda i: (0, 0)),
          pl.BlockSpec((HID_DIM, OUT_DIM), lambda i: (0, 0)),
          pl.BlockSpec((HID_DIM, OUT_DIM), lambda i: (0, 0)),
      ],
      out_specs=[
          pl.BlockSpec((NC, ROW_BLK, H2), lambda i: (0, i, 0)),
          pl.BlockSpec((ROW_BLK, OUT_DIM), lambda i: (i, 0)),
      ],
      out_shape=[
          jax.ShapeDtypeStruct((NC, N_NODES, H2), jnp.float32),
          jax.ShapeDtypeStruct((N_NODES, OUT_DIM), jnp.float32),
      ],
  )(p, cnt, xr, b1, w2l, w2r)


def _stage_e_body(q_ref, c_ref, hr_ref, b2_ref, out_ref):
  cnt = c_ref[:, 0:1]
  rcp = 1.0 / jnp.maximum(cnt, 1.0)
  mean = jnp.concatenate([q_ref[0], q_ref[1]], axis=1) * rcp
  out_ref[...] = mean + hr_ref[...] + b2_ref[...]


def _stage_e(q, cnt, hr, b2):
  return pl.pallas_call(
      _stage_e_body,
      grid=(GRID,),
      in_specs=[
          pl.BlockSpec((NC, ROW_BLK, H2), lambda i: (0, i, 0)),
          pl.BlockSpec((ROW_BLK, CNT_W), lambda i: (i, 0)),
          pl.BlockSpec((ROW_BLK, OUT_DIM), lambda i: (i, 0)),
          pl.BlockSpec((1, OUT_DIM), lambda i: (0, 0)),
      ],
      out_specs=pl.BlockSpec((ROW_BLK, OUT_DIM), lambda i: (i, 0)),
      out_shape=jax.ShapeDtypeStruct((N_NODES, OUT_DIM), jnp.float32),
  )(q, cnt, hr, b2)


@jax.jit
def kernel(x, edge_index, W1_l, W1_r, b1, W2_l, W2_r, b2):
  src = edge_index[0].astype(jnp.int32)
  dst = edge_index[1].astype(jnp.int32)
  pad = EDGES_PAD - N_EDGES
  src_p = jnp.concatenate([src, jnp.zeros((pad,), jnp.int32)])
  src3 = jnp.stack([src_p, src_p + N_NODES]).reshape(NC, NS, IBLOCKS, EDGE_BLK)
  dst3 = jnp.concatenate([dst, jnp.full((pad,), N_NODES, jnp.int32)])
  dst3 = dst3.reshape(NS, IBLOCKS, EDGE_BLK)

  z1 = jnp.zeros((ZCHUNK, H1), jnp.float32)
  z2 = jnp.zeros((ZCHUNK, H2), jnp.float32)
  zc = jnp.zeros((ZCHUNK, CNT_W), jnp.float32)
  ones = jnp.ones((EDGE_BLK, CNT_W), jnp.float32)

  y1, xr = _stage_a(x, W1_l, W1_r)
  p1, c1 = _seg_sum_l1(y1.reshape(NC * N_NODES, H1), src3, dst3, z1, zc, ones)
  y2, hr = _stage_c(p1, c1[:N_NODES], xr, b1.reshape(1, HID_DIM), W2_l, W2_r)
  (q2,) = _seg_sum_l2(y2.reshape(NC * N_NODES, H2), src3, dst3, z2)
  out = _stage_e(q2, c1[:N_NODES], hr, b2.reshape(1, OUT_DIM))
  return out
